# baseline (speedup 1.0000x reference)
import jax
import jax.numpy as jnp
from jax import lax
from jax.experimental import pallas as pl
from jax.experimental.pallas import tpu as pltpu
from jax.experimental.pallas import tpu_sc as plsc

N = 4096
BATCH = 16384
WEIGHT = 0.999

NC = 2
NS = 16
NW = NC * NS

ROWS_PER_W = BATCH // NW
R = 32
G = ROWS_PER_W // R
K = G // 2

HIST_W = 4608
DUMP_BIN = N

_M8 = 0x00FF00FF
_M16 = 0x0000FFFF


def _mesh():
    return plsc.VectorSubcoreMesh(
        core_axis_name="c", subcore_axis_name="s",
        num_cores=NC, num_subcores=NS)


def _row_hist_body(mask_hbm, hist_hbm, buf0, buf1, tbuf, hist, sem0, sem1):
    wid = lax.axis_index("s") * NC + lax.axis_index("c")
    row0 = wid * ROWS_PER_W

    def zero_body(i, _):
        hist[pl.ds(i * 16, 16)] = jnp.zeros((16,), jnp.int32)
        return 0

    lax.fori_loop(0, HIST_W // 16, zero_body, 0)

    def copy(chunk, buf, sem):
        return pltpu.make_async_copy(
            mask_hbm.at[pl.ds(row0 + chunk * R, R), :], buf, sem)

    def process(buf, chunk):
        def row_body(rr, _):
            acc = jnp.zeros((16,), jnp.int32)
            for j in range(64):
                x = buf[rr, pl.ds(j * 64, 64)]
                acc = acc + plsc.bitcast(x, jnp.int32)
            t = (acc & _M8) + ((acc >> 8) & _M8)
            tbuf[pl.ds((chunk * R + rr) * 16, 16)] = t
            return 0

        lax.fori_loop(0, R, row_body, 0)

    copy(0, buf0, sem0).start()

    def pair_body(k, _):
        a = 2 * k
        copy(a + 1, buf1, sem1).start()
        copy(a, buf0, sem0).wait()
        process(buf0, a)

        @pl.when(k < K - 1)
        def _():
            copy(a + 2, buf0, sem0).start()

        copy(a + 1, buf1, sem1).wait()
        process(buf1, a + 1)
        return 0

    lax.fori_loop(0, K, pair_body, 0)

    iota16 = lax.iota(jnp.int32, 16)

    def group_body(g, _):
        colbase = g * 256 + iota16 * 16
        t_sum = jnp.zeros((16,), jnp.int32)
        for l in range(16):
            t_sum = t_sum + plsc.load_gather(tbuf, [colbase + l])
        lens = (t_sum & _M16) + (t_sum >> 16)
        bins = jnp.where(lens == 0, DUMP_BIN, lens - 1)
        cnt, last = plsc.scan_count(bins)
        plsc.addupdate_scatter(hist, [bins], cnt, mask=last)
        return 0

    lax.fori_loop(0, ROWS_PER_W // 16, group_body, 0)

    pltpu.sync_copy(hist, hist_hbm.at[wid])


def _combine_body(hist_hbm, prob_hbm, out_hbm, hb, pb, ob, sem):
    wid = lax.axis_index("s") * NC + lax.axis_index("c")
    col0 = wid * (N // NW)

    for r in range(NW):
        pltpu.make_async_copy(
            hist_hbm.at[r, pl.ds(col0, N // NW)], hb.at[r], sem).start()
    pltpu.sync_copy(prob_hbm.at[pl.ds(col0, N // NW)], pb)
    for r in range(NW):
        pltpu.make_async_copy(
            hist_hbm.at[r, pl.ds(col0, N // NW)], hb.at[r], sem).wait()

    w = jnp.float32(WEIGHT)
    one_minus_w = jnp.float32(1.0 - WEIGHT)
    inv_batch = jnp.float32(1.0 / BATCH)
    for jj in range(N // NW // 16):
        c = jnp.zeros((16,), jnp.int32)
        for r in range(NW):
            c = c + hb[r, pl.ds(jj * 16, 16)]
        batch_prob = c.astype(jnp.float32) * inv_batch
        ob[pl.ds(jj * 16, 16)] = (
            w * pb[pl.ds(jj * 16, 16)] + one_minus_w * batch_prob)
    pltpu.sync_copy(ob, out_hbm.at[pl.ds(col0, N // NW)])


def kernel(n_elements_prob, mask):
    assert mask.shape == (BATCH, N) and mask.dtype == jnp.bool_

    hist_all = pl.kernel(
        _row_hist_body,
        out_type=jax.ShapeDtypeStruct((NW, HIST_W), jnp.int32),
        mesh=_mesh(),
        compiler_params=pltpu.CompilerParams(needs_layout_passes=False),
        scratch_types=[
            pltpu.VMEM((R, N), jnp.int8),
            pltpu.VMEM((R, N), jnp.int8),
            pltpu.VMEM((ROWS_PER_W * 16,), jnp.int32),
            pltpu.VMEM((HIST_W,), jnp.int32),
            pltpu.SemaphoreType.DMA,
            pltpu.SemaphoreType.DMA,
        ],
    )(mask.astype(jnp.int8))

    new_prob = pl.kernel(
        _combine_body,
        out_type=jax.ShapeDtypeStruct((N,), jnp.float32),
        mesh=_mesh(),
        scratch_types=[
            pltpu.VMEM((NW, N // NW), jnp.int32),
            pltpu.VMEM((N // NW,), jnp.float32),
            pltpu.VMEM((N // NW,), jnp.float32),
            pltpu.SemaphoreType.DMA,
        ],
    )(hist_all, n_elements_prob)

    return new_prob
